# Initial kernel scaffold; baseline (speedup 1.0000x reference)
import jax
import jax.numpy as jnp
from jax.experimental import pallas as pl


def kernel(token_ids, embeddings):
    raise NotImplementedError("write your pallas kernel here")


# speedup vs baseline: 1.1924x; 1.1924x over previous
import functools

import jax
import jax.numpy as jnp
from jax import lax
from jax.experimental import pallas as pl
from jax.experimental.pallas import tpu as pltpu
from jax.experimental.pallas import tpu_sc as plsc

NUM_EMB = 1000000
DIM = 32
BATCH = 16384
HIST = 50
TOTAL = BATCH * HIST
NC = 2
NS = 16
NW = NC * NS
PER_W = TOTAL // NW
CHUNK = 128
NCHUNK = PER_W // CHUNK


def _body(tok_hbm, table_hbm, out_hbm, idx_v, rows_v, gsem):
    wid = lax.axis_index("s") * NC + lax.axis_index("c")
    pltpu.sync_copy(tok_hbm.at[wid], idx_v)

    def step(g, _):
        pltpu.async_copy(table_hbm.at[idx_v.at[g]], rows_v, gsem).wait()
        pltpu.sync_copy(rows_v, out_hbm.at[wid, pl.ds(g * CHUNK, CHUNK)])
        return _

    lax.fori_loop(0, NCHUNK, step, 0, unroll=False)


@functools.lru_cache(maxsize=1)
def _build():
    mesh = plsc.VectorSubcoreMesh(core_axis_name="c", subcore_axis_name="s")
    return pl.kernel(
        _body,
        out_type=jax.ShapeDtypeStruct((NW, PER_W, DIM), jnp.float32),
        mesh=mesh,
        scratch_types=[
            pltpu.VMEM((NCHUNK, CHUNK), jnp.int32),
            pltpu.VMEM((CHUNK, DIM), jnp.float32),
            pltpu.SemaphoreType.DMA,
        ],
        compiler_params=pltpu.CompilerParams(use_tc_tiling_on_sc=False),
    )


def kernel(token_ids, embeddings):
    tok = token_ids.astype(jnp.int32).reshape(NW, NCHUNK, CHUNK)
    out = _build()(tok, embeddings)
    return out.reshape(BATCH, HIST, DIM)


# speedup vs baseline: 1.3069x; 1.0960x over previous
import functools

import jax
import jax.numpy as jnp
from jax import lax
from jax.experimental import pallas as pl
from jax.experimental.pallas import tpu as pltpu
from jax.experimental.pallas import tpu_sc as plsc

NUM_EMB = 1000000
DIM = 32
BATCH = 16384
HIST = 50
TOTAL = BATCH * HIST
NC = 2
NS = 16
NW = NC * NS
PER_W = TOTAL // NW
CHUNK = 128
SZ = 10
GRP = SZ * CHUNK
NGRP = PER_W // GRP


def _body(tok_hbm, table_hbm, out_hbm, idx_v, rows0, rows1, gsem, osem):
    wid = lax.axis_index("s") * NC + lax.axis_index("c")
    pltpu.sync_copy(tok_hbm.at[wid], idx_v)
    bufs = (rows0, rows1)

    def out_dst(g):
        return out_hbm.at[wid, pl.ds(g * GRP, GRP)]

    def pair(i, _):
        for p in range(2):
            buf = bufs[p]
            @pl.when(i > 0)
            def _wait():
                pltpu.make_async_copy(buf, out_dst(0), osem).wait()

            g = 2 * i + p
            pltpu.async_copy(
                table_hbm.at[idx_v.at[pl.ds(g * GRP, GRP)]], buf, gsem
            ).wait()
            pltpu.async_copy(buf, out_dst(g), osem)
        return _

    lax.fori_loop(0, NGRP // 2, pair, 0, unroll=False)
    for p in range(2):
        pltpu.make_async_copy(bufs[p], out_dst(0), osem).wait()


@functools.lru_cache(maxsize=1)
def _build():
    mesh = plsc.VectorSubcoreMesh(core_axis_name="c", subcore_axis_name="s")
    return pl.kernel(
        _body,
        out_type=jax.ShapeDtypeStruct((NW, PER_W, DIM), jnp.float32),
        mesh=mesh,
        scratch_types=[
            pltpu.VMEM((PER_W,), jnp.int32),
            pltpu.VMEM((GRP, DIM), jnp.float32),
            pltpu.VMEM((GRP, DIM), jnp.float32),
            pltpu.SemaphoreType.DMA,
            pltpu.SemaphoreType.DMA,
        ],
        compiler_params=pltpu.CompilerParams(use_tc_tiling_on_sc=False),
    )


def kernel(token_ids, embeddings):
    tok = token_ids.astype(jnp.int32).reshape(NW, PER_W)
    out = _build()(tok, embeddings)
    return out.reshape(BATCH, HIST, DIM)


# speedup vs baseline: 1.3105x; 1.0028x over previous
import functools

import jax
import jax.numpy as jnp
from jax import lax
from jax.experimental import pallas as pl
from jax.experimental.pallas import tpu as pltpu
from jax.experimental.pallas import tpu_sc as plsc

NUM_EMB = 1000000
DIM = 32
BATCH = 16384
HIST = 50
TOTAL = BATCH * HIST
NC = 2
NS = 16
NW = NC * NS
PER_W = TOTAL // NW
GRP = 640
NGRP = PER_W // GRP
NB = 4


def _body(tok_hbm, table_hbm, out_hbm, idx_v, *rest):
    bufs = rest[:NB]
    gsems = rest[NB:2 * NB]
    osems = rest[2 * NB:3 * NB]
    wid = lax.axis_index("s") * NC + lax.axis_index("c")
    pltpu.sync_copy(tok_hbm.at[wid], idx_v)

    def fire_gather(g, b):
        pltpu.async_copy(
            table_hbm.at[idx_v.at[pl.ds(g * GRP, GRP)]], bufs[b], gsems[b]
        )

    def wait_gather(b):
        pltpu.make_async_copy(
            table_hbm.at[pl.ds(0, GRP)], bufs[b], gsems[b]
        ).wait()

    def out_dst(g):
        return out_hbm.at[wid, pl.ds(g * GRP, GRP)]

    for b in range(NB):
        fire_gather(b, b)

    def step(i, _):
        for b in range(NB):
            g = NB * i + b
            wait_gather(b)
            pltpu.async_copy(bufs[b], out_dst(g), osems[b])

            @pl.when(g + NB < NGRP)
            def _next():
                pltpu.make_async_copy(bufs[b], out_dst(0), osems[b]).wait()
                fire_gather(g + NB, b)
        return _

    lax.fori_loop(0, NGRP // NB, step, 0, unroll=False)
    for b in range(NB):
        pltpu.make_async_copy(bufs[b], out_dst(0), osems[b]).wait()


@functools.lru_cache(maxsize=1)
def _build():
    mesh = plsc.VectorSubcoreMesh(core_axis_name="c", subcore_axis_name="s")
    return pl.kernel(
        _body,
        out_type=jax.ShapeDtypeStruct((NW, PER_W, DIM), jnp.float32),
        mesh=mesh,
        scratch_types=(
            [pltpu.VMEM((PER_W,), jnp.int32)]
            + [pltpu.VMEM((GRP, DIM), jnp.float32) for _ in range(NB)]
            + [pltpu.SemaphoreType.DMA for _ in range(2 * NB)]
        ),
        compiler_params=pltpu.CompilerParams(use_tc_tiling_on_sc=False),
    )


def kernel(token_ids, embeddings):
    tok = token_ids.astype(jnp.int32).reshape(NW, PER_W)
    out = _build()(tok, embeddings)
    return out.reshape(BATCH, HIST, DIM)


# speedup vs baseline: 1.5399x; 1.1751x over previous
import functools

import jax
import jax.numpy as jnp
from jax import lax
from jax.experimental import pallas as pl
from jax.experimental.pallas import tpu as pltpu
from jax.experimental.pallas import tpu_sc as plsc

NUM_EMB = 1000000
DIM = 32
BATCH = 16384
HIST = 50
NC = 2
NS = 16
NW = NC * NS
TBW = BATCH // 128 // NW
NBLK = HIST * TBW
L = 16


def _transpose_block(rbuf, tbuf):
    for td in range(4):
        for r in range(8):
            d_idx = jnp.full((L,), 8 * td + r, jnp.int32)
            for k in range(8):
                c_idx = lax.iota(jnp.int32, L) + L * k
                vals = plsc.load_gather(rbuf, [c_idx, d_idx])
                tbuf[td, r, pl.ds(L * k, L)] = vals


def _body(tok_hbm, table_hbm, out_hbm, idx_v, r0, r1, t0, t1, gs0, gs1,
          os0, os1):
    wid = lax.axis_index("s") * NC + lax.axis_index("c")
    rbufs, tbufs = (r0, r1), (t0, t1)
    gsems, osems = (gs0, gs1), (os0, os1)
    pltpu.sync_copy(tok_hbm.at[:, pl.ds(512 * wid, 512)], idx_v)

    def idx_of(g):
        return idx_v.at[g // TBW, pl.ds(128 * (g % TBW), 128)]

    def fire_gather(g, s):
        pltpu.async_copy(table_hbm.at[idx_of(g)], rbufs[s], gsems[s])

    def wait_gather(s):
        pltpu.make_async_copy(
            table_hbm.at[pl.ds(0, 128)], rbufs[s], gsems[s]
        ).wait()

    def fire_out(g, s):
        h, tb = g // TBW, 4 * wid + (g % TBW)
        for td in range(4):
            pltpu.async_copy(tbufs[s].at[td], out_hbm.at[h, td, tb], osems[s])

    def wait_out(s):
        for td in range(4):
            pltpu.make_async_copy(
                tbufs[s].at[td], out_hbm.at[0, td, 0], osems[s]
            ).wait()

    fire_gather(0, 0)
    fire_gather(1, 1)

    def step(g, _):
        for s in range(2):
            @pl.when(g >= 2)
            def _free():
                wait_out(s)
            wait_gather(s)
            _transpose_block(rbufs[s], tbufs[s])

            @pl.when(g + 2 + s < NBLK)
            def _refill():
                fire_gather(g + 2 + s, s)
            fire_out(g + s, s)
        return _

    lax.fori_loop(0, NBLK // 2, lambda i, c: step(2 * i, c), 0, unroll=False)
    wait_out(0)
    wait_out(1)


@functools.lru_cache(maxsize=1)
def _build():
    mesh = plsc.VectorSubcoreMesh(core_axis_name="c", subcore_axis_name="s")
    return pl.kernel(
        _body,
        out_type=jax.ShapeDtypeStruct((HIST, 4, 128, 8, 128), jnp.float32),
        mesh=mesh,
        scratch_types=(
            [pltpu.VMEM((HIST, 512), jnp.int32)]
            + [pltpu.VMEM((128, DIM), jnp.float32) for _ in range(2)]
            + [pltpu.VMEM((4, 8, 128), jnp.float32) for _ in range(2)]
            + [pltpu.SemaphoreType.DMA for _ in range(4)]
        ),
        compiler_params=pltpu.CompilerParams(
            use_tc_tiling_on_sc=False, needs_layout_passes=False
        ),
    )


def kernel(token_ids, embeddings):
    tok_t = jnp.transpose(token_ids)
    x = _build()(tok_t, embeddings)
    return x.transpose(2, 4, 0, 1, 3).reshape(BATCH, HIST, DIM)


# speedup vs baseline: 2.8190x; 1.8307x over previous
import functools

import jax
import jax.numpy as jnp
from jax import lax
from jax.experimental import pallas as pl
from jax.experimental.pallas import tpu as pltpu
from jax.experimental.pallas import tpu_sc as plsc

NUM_EMB = 1000000
DIM = 32
BATCH = 16384
HIST = 50
NC = 2
NS = 16
NW = NC * NS
TBW = BATCH // 128 // NW
NBLK = HIST * TBW
L = 16


def _transpose_block(rbuf, tbuf):
    lanes = lax.iota(jnp.int32, L)
    td_lo = lanes // 8
    td_hi = td_lo + 2
    r_pat = lanes % 8

    def col(c, carry):
        v0 = rbuf[c, pl.ds(0, L)]
        v1 = rbuf[c, pl.ds(L, L)]
        cs = jnp.full((L,), c, jnp.int32)
        plsc.store_scatter(tbuf, [td_lo, r_pat, cs], v0)
        plsc.store_scatter(tbuf, [td_hi, r_pat, cs], v1)
        return carry

    lax.fori_loop(0, 128, col, 0, unroll=8)


def _body(tok_hbm, table_hbm, out_hbm, idx_v, r0, r1, t0, t1, gs0, gs1,
          os0, os1):
    wid = lax.axis_index("s") * NC + lax.axis_index("c")
    rbufs, tbufs = (r0, r1), (t0, t1)
    gsems, osems = (gs0, gs1), (os0, os1)
    pltpu.sync_copy(tok_hbm.at[:, pl.ds(512 * wid, 512)], idx_v)

    def idx_of(g):
        return idx_v.at[g // TBW, pl.ds(128 * (g % TBW), 128)]

    def fire_gather(g, s):
        pltpu.async_copy(table_hbm.at[idx_of(g)], rbufs[s], gsems[s])

    def wait_gather(s):
        pltpu.make_async_copy(
            table_hbm.at[pl.ds(0, 128)], rbufs[s], gsems[s]
        ).wait()

    def fire_out(g, s):
        h, tb = g // TBW, 4 * wid + (g % TBW)
        for td in range(4):
            pltpu.async_copy(
                tbufs[s].at[td, :, pl.ds(0, 128)], out_hbm.at[h, td, tb],
                osems[s],
            )

    def wait_out(s):
        for td in range(4):
            pltpu.make_async_copy(
                tbufs[s].at[td, :, pl.ds(0, 128)], out_hbm.at[0, td, 0],
                osems[s],
            ).wait()

    fire_gather(0, 0)
    fire_gather(1, 1)

    def step(g, _):
        for s in range(2):
            @pl.when(g >= 2)
            def _free():
                wait_out(s)
            wait_gather(s)
            _transpose_block(rbufs[s], tbufs[s])

            @pl.when(g + 2 + s < NBLK)
            def _refill():
                fire_gather(g + 2 + s, s)
            fire_out(g + s, s)
        return _

    lax.fori_loop(0, NBLK // 2, lambda i, c: step(2 * i, c), 0, unroll=False)
    wait_out(0)
    wait_out(1)


@functools.lru_cache(maxsize=1)
def _build():
    mesh = plsc.VectorSubcoreMesh(core_axis_name="c", subcore_axis_name="s")
    return pl.kernel(
        _body,
        out_type=jax.ShapeDtypeStruct((HIST, 4, 128, 8, 128), jnp.float32),
        mesh=mesh,
        scratch_types=(
            [pltpu.VMEM((HIST, 512), jnp.int32)]
            + [pltpu.VMEM((128, DIM), jnp.float32) for _ in range(2)]
            + [pltpu.VMEM((4, 8, 129), jnp.float32) for _ in range(2)]
            + [pltpu.SemaphoreType.DMA for _ in range(4)]
        ),
        compiler_params=pltpu.CompilerParams(
            use_tc_tiling_on_sc=False, needs_layout_passes=False
        ),
    )


def kernel(token_ids, embeddings):
    tok_t = jnp.transpose(token_ids)
    x = _build()(tok_t, embeddings)
    return x.transpose(2, 4, 0, 1, 3).reshape(BATCH, HIST, DIM)
